# baseline (speedup 1.0000x reference)
import jax
import jax.numpy as jnp
from jax import lax
from jax.experimental import pallas as pl
from jax.experimental.pallas import tpu as pltpu
from jax.experimental.pallas import tpu_sc as plsc

N_NODES = 10000
D = 128

NC = 2
NS = 16
CHUNK = 128
PHASE_CHUNKS = 16
CORE_PHASES = (10, 0)
CORE_CHUNKS = tuple(p * PHASE_CHUNKS for p in CORE_PHASES)
NBUF = 2
E_PAD = NS * CHUNK * sum(CORE_CHUNKS)
ROWS_PER_TILE = 632
ACC_ROWS = NS * ROWS_PER_TILE
N_DUMMY = 112


def _sc_kernel_body(x_hbm, src0_hbm, dst0_hbm,
                    zeros_hbm, out_hbm, src_v, dst_v, ring, acc, gsem, ssem):
  c = lax.axis_index("c")
  s = lax.axis_index("s")

  slab = pl.ds(s * ROWS_PER_TILE, ROWS_PER_TILE)

  @pl.when(c == 0)
  def _init():
    pltpu.sync_copy(zeros_hbm.at[slab], acc.at[slab])
  plsc.subcore_barrier()

  def gather_start(ci, b):
    pltpu.async_copy(x_hbm.at[src_v.at[ci]], ring.at[b], gsem.at[b])

  def gather_wait(ci, b):
    pltpu.make_async_copy(x_hbm.at[src_v.at[ci]], ring.at[b], gsem.at[b]).wait()

  def scatter_start(ci, b):
    pltpu.async_copy(ring.at[b], acc.at[dst_v.at[ci]], ssem.at[b], add=True)

  def scatter_wait(ci, b):
    pltpu.make_async_copy(ring.at[b], acc.at[dst_v.at[ci]], ssem.at[b]).wait()

  def pipeline(src_hbm, dst_hbm, n_phases):
    for phase in range(n_phases):
      pblk = pl.ds(phase * PHASE_CHUNKS, PHASE_CHUNKS)
      pltpu.sync_copy(src_hbm.at[s, pblk], src_v)
      pltpu.sync_copy(dst_hbm.at[s, pblk], dst_v)

      for b in range(NBUF):
        gather_start(b, b)

      n_groups = PHASE_CHUNKS // NBUF

      @pl.loop(0, n_groups - 1)
      def _group(g):
        base = g * NBUF
        for b in range(NBUF):
          gather_wait(base + b, b)
          scatter_start(base + b, b)
        for b in range(NBUF):
          scatter_wait(base + b, b)
          gather_start(base + NBUF + b, b)

      last = (n_groups - 1) * NBUF
      for b in range(NBUF):
        gather_wait(last + b, b)
        scatter_start(last + b, b)
      for b in range(NBUF):
        scatter_wait(last + b, b)

  @pl.when(c == 0)
  def _core0():
    pipeline(src0_hbm, dst0_hbm, CORE_PHASES[0])

  plsc.subcore_barrier()

  @pl.when(c == 0)
  def _writeout():
    pltpu.sync_copy(acc.at[slab], out_hbm.at[0, slab])


def _segment_accumulate(x, src0, dst0, zeros):
  mesh = plsc.VectorSubcoreMesh(
      core_axis_name="c", subcore_axis_name="s", num_cores=NC, num_subcores=NS)
  kern = pl.kernel(
      _sc_kernel_body,
      out_type=jax.ShapeDtypeStruct((1, ACC_ROWS, D), jnp.float32),
      mesh=mesh,
      scratch_types=[
          pltpu.VMEM((PHASE_CHUNKS, CHUNK), jnp.int32),
          pltpu.VMEM((PHASE_CHUNKS, CHUNK), jnp.int32),
          pltpu.VMEM((NBUF, CHUNK, D), jnp.float32),
          pltpu.VMEM_SHARED((ACC_ROWS, D), jnp.float32),
          pltpu.SemaphoreType.DMA((NBUF,)),
          pltpu.SemaphoreType.DMA((NBUF,)),
      ],
  )
  return kern(x, src0, dst0, zeros)


def _mm_body(p_ref, x_ref, w_ref, ws_ref, o_ref):
  agg = p_ref[0]
  o_ref[...] = (jnp.dot(agg, w_ref[...], preferred_element_type=jnp.float32)
                + jnp.dot(x_ref[...], ws_ref[...], preferred_element_type=jnp.float32))


def _finish(partial, x, weight, weight_self):
  blk = 1000
  grid = (N_NODES // blk,)
  return pl.pallas_call(
      _mm_body,
      grid=grid,
      in_specs=[
          pl.BlockSpec((1, blk, D), lambda i: (0, i, 0)),
          pl.BlockSpec((blk, D), lambda i: (i, 0)),
          pl.BlockSpec((D, D), lambda i: (0, 0)),
          pl.BlockSpec((D, D), lambda i: (0, 0)),
      ],
      out_specs=pl.BlockSpec((blk, D), lambda i: (i, 0)),
      out_shape=jax.ShapeDtypeStruct((N_NODES, D), jnp.float32),
  )(partial, x, weight, weight_self)


@jax.jit
def kernel(x, edge_index, weight, weight_self):
  n_edges = edge_index.shape[1]
  pad = E_PAD - n_edges
  src = jnp.concatenate([edge_index[0], jnp.zeros((pad,), jnp.int32)])
  pad_dst = N_NODES + (jnp.arange(pad, dtype=jnp.int32) % N_DUMMY)
  dst = jnp.concatenate([edge_index[1], pad_dst])
  src0 = src.reshape(NS, CORE_CHUNKS[0], CHUNK)
  dst0 = dst.reshape(NS, CORE_CHUNKS[0], CHUNK)
  zeros = jnp.zeros((ACC_ROWS, D), jnp.float32)

  partial = _segment_accumulate(x, src0, dst0, zeros)
  return _finish(partial[:, :N_NODES, :], x, weight, weight_self)


# speedup vs baseline: 3.7427x; 3.7427x over previous
import jax
import jax.numpy as jnp
from jax import lax
from jax.experimental import pallas as pl
from jax.experimental.pallas import tpu as pltpu
from jax.experimental.pallas import tpu_sc as plsc

N_NODES = 10000
D = 128

NC = 2
NS = 16
CHUNK = 128
PHASE_CHUNKS = 16
CORE_PHASES = (5, 5)
CORE_CHUNKS = tuple(p * PHASE_CHUNKS for p in CORE_PHASES)
NBUF = 2
E_PAD = NS * CHUNK * sum(CORE_CHUNKS)
ROWS_PER_TILE = 632
ACC_ROWS = NS * ROWS_PER_TILE
N_DUMMY = 112


def _sc_kernel_body(x_hbm, src_hbm, dst_hbm,
                    zeros_hbm, out_hbm, src_v, dst_v, ring, acc, gsem, ssem):
  c = lax.axis_index("c")
  s = lax.axis_index("s")
  wid = c * NS + s

  slab = pl.ds(s * ROWS_PER_TILE, ROWS_PER_TILE)
  pltpu.sync_copy(zeros_hbm.at[slab], acc.at[slab])
  plsc.subcore_barrier()

  def gather_start(ci, b):
    pltpu.async_copy(x_hbm.at[src_v.at[ci]], ring.at[b], gsem.at[b])

  def gather_wait(ci, b):
    pltpu.make_async_copy(x_hbm.at[src_v.at[ci]], ring.at[b], gsem.at[b]).wait()

  def scatter_start(ci, b):
    pltpu.async_copy(ring.at[b], acc.at[dst_v.at[ci]], ssem.at[b], add=True)

  def scatter_wait(ci, b):
    pltpu.make_async_copy(ring.at[b], acc.at[dst_v.at[ci]], ssem.at[b]).wait()

  def pipeline(n_phases):
    for phase in range(n_phases):
      pblk = pl.ds(phase * PHASE_CHUNKS, PHASE_CHUNKS)
      pltpu.sync_copy(src_hbm.at[wid, pblk], src_v)
      pltpu.sync_copy(dst_hbm.at[wid, pblk], dst_v)

      for b in range(NBUF):
        gather_start(b, b)

      n_groups = PHASE_CHUNKS // NBUF

      @pl.loop(0, n_groups - 1)
      def _group(g):
        base = g * NBUF
        for b in range(NBUF):
          gather_wait(base + b, b)
          scatter_start(base + b, b)
        for b in range(NBUF):
          scatter_wait(base + b, b)
          gather_start(base + NBUF + b, b)

      last = (n_groups - 1) * NBUF
      for b in range(NBUF):
        gather_wait(last + b, b)
        scatter_start(last + b, b)
      for b in range(NBUF):
        scatter_wait(last + b, b)

  pipeline(CORE_PHASES[0])

  plsc.subcore_barrier()

  pltpu.sync_copy(acc.at[slab], out_hbm.at[c, slab])


def _segment_accumulate(x, src_blocks, dst_blocks, zeros):
  mesh = plsc.VectorSubcoreMesh(
      core_axis_name="c", subcore_axis_name="s", num_cores=NC, num_subcores=NS)
  kern = pl.kernel(
      _sc_kernel_body,
      out_type=jax.ShapeDtypeStruct((NC, ACC_ROWS, D), jnp.float32),
      mesh=mesh,
      scratch_types=[
          pltpu.VMEM((PHASE_CHUNKS, CHUNK), jnp.int32),
          pltpu.VMEM((PHASE_CHUNKS, CHUNK), jnp.int32),
          pltpu.VMEM((NBUF, CHUNK, D), jnp.float32),
          pltpu.VMEM_SHARED((ACC_ROWS, D), jnp.float32),
          pltpu.SemaphoreType.DMA((NBUF,)),
          pltpu.SemaphoreType.DMA((NBUF,)),
      ],
  )
  return kern(x, src_blocks, dst_blocks, zeros)


def _mm_body(p_ref, x_ref, w_ref, ws_ref, o_ref):
  agg = p_ref[0]
  for i in range(1, NC):
    agg = agg + p_ref[i]
  o_ref[...] = (jnp.dot(agg, w_ref[...], preferred_element_type=jnp.float32)
                + jnp.dot(x_ref[...], ws_ref[...], preferred_element_type=jnp.float32))


def _finish(partial, x, weight, weight_self):
  blk = 1000
  grid = (N_NODES // blk,)
  return pl.pallas_call(
      _mm_body,
      grid=grid,
      in_specs=[
          pl.BlockSpec((NC, blk, D), lambda i: (0, i, 0)),
          pl.BlockSpec((blk, D), lambda i: (i, 0)),
          pl.BlockSpec((D, D), lambda i: (0, 0)),
          pl.BlockSpec((D, D), lambda i: (0, 0)),
      ],
      out_specs=pl.BlockSpec((blk, D), lambda i: (i, 0)),
      out_shape=jax.ShapeDtypeStruct((N_NODES, D), jnp.float32),
  )(partial, x, weight, weight_self)


@jax.jit
def kernel(x, edge_index, weight, weight_self):
  n_edges = edge_index.shape[1]
  pad = E_PAD - n_edges
  pad_src = jnp.arange(pad, dtype=jnp.int32) % N_NODES
  src = jnp.concatenate([edge_index[0], pad_src])
  pad_dst = N_NODES + (jnp.arange(pad, dtype=jnp.int32) % N_DUMMY)
  dst = jnp.concatenate([edge_index[1], pad_dst])
  src_blocks = src.reshape(NC * NS, CORE_CHUNKS[0], CHUNK)
  dst_blocks = dst.reshape(NC * NS, CORE_CHUNKS[0], CHUNK)
  zeros = jnp.zeros((ACC_ROWS, D), jnp.float32)

  partial = _segment_accumulate(x, src_blocks, dst_blocks, zeros)
  return _finish(partial[:, :N_NODES, :], x, weight, weight_self)


# speedup vs baseline: 3.9603x; 1.0581x over previous
import jax
import jax.numpy as jnp
from jax import lax
from jax.experimental import pallas as pl
from jax.experimental.pallas import tpu as pltpu
from jax.experimental.pallas import tpu_sc as plsc

N_NODES = 10000
D = 128

NC = 2
NS = 16
CHUNK = 128
PHASE_CHUNKS = 16
CORE_PHASES = (5, 5)
CORE_CHUNKS = tuple(p * PHASE_CHUNKS for p in CORE_PHASES)
NBUF = 2
E_PAD = NS * CHUNK * sum(CORE_CHUNKS)
ROWS_PER_TILE = 632
ACC_ROWS = NS * ROWS_PER_TILE
LAST_ROWS = N_NODES - (NS - 1) * ROWS_PER_TILE
N_DUMMY = 112


def _sc_kernel_body(x_hbm, src_hbm, dst_hbm,
                    zeros_hbm, out_hbm, src_v, dst_v, ring, acc, gsem, ssem):
  c = lax.axis_index("c")
  s = lax.axis_index("s")
  wid = c * NS + s

  slab = pl.ds(s * ROWS_PER_TILE, ROWS_PER_TILE)
  pltpu.sync_copy(zeros_hbm.at[slab], acc.at[slab])
  plsc.subcore_barrier()

  def gather_start(ci, b):
    pltpu.async_copy(x_hbm.at[src_v.at[ci]], ring.at[b], gsem.at[b])

  def gather_wait(ci, b):
    pltpu.make_async_copy(x_hbm.at[src_v.at[ci]], ring.at[b], gsem.at[b]).wait()

  def scatter_start(ci, b):
    pltpu.async_copy(ring.at[b], acc.at[dst_v.at[ci]], ssem.at[b], add=True)

  def scatter_wait(ci, b):
    pltpu.make_async_copy(ring.at[b], acc.at[dst_v.at[ci]], ssem.at[b]).wait()

  def pipeline(n_phases):
    for phase in range(n_phases):
      pblk = pl.ds(phase * PHASE_CHUNKS, PHASE_CHUNKS)
      pltpu.sync_copy(src_hbm.at[wid, pblk], src_v)
      pltpu.sync_copy(dst_hbm.at[wid, pblk], dst_v)

      for b in range(NBUF):
        gather_start(b, b)

      n_groups = PHASE_CHUNKS // NBUF

      @pl.loop(0, n_groups - 1)
      def _group(g):
        base = g * NBUF
        for b in range(NBUF):
          gather_wait(base + b, b)
          scatter_start(base + b, b)
        for b in range(NBUF):
          scatter_wait(base + b, b)
          gather_start(base + NBUF + b, b)

      last = (n_groups - 1) * NBUF
      for b in range(NBUF):
        gather_wait(last + b, b)
        scatter_start(last + b, b)
      for b in range(NBUF):
        scatter_wait(last + b, b)

  pipeline(CORE_PHASES[0])

  plsc.subcore_barrier()

  @pl.when(s < NS - 1)
  def _writeout_full():
    pltpu.sync_copy(acc.at[slab], out_hbm.at[c, slab])

  @pl.when(s == NS - 1)
  def _writeout_last():
    short = pl.ds((NS - 1) * ROWS_PER_TILE, LAST_ROWS)
    pltpu.sync_copy(acc.at[short], out_hbm.at[c, short])


def _segment_accumulate(x, src_blocks, dst_blocks, zeros):
  mesh = plsc.VectorSubcoreMesh(
      core_axis_name="c", subcore_axis_name="s", num_cores=NC, num_subcores=NS)
  kern = pl.kernel(
      _sc_kernel_body,
      out_type=jax.ShapeDtypeStruct((NC, N_NODES, D), jnp.float32),
      mesh=mesh,
      scratch_types=[
          pltpu.VMEM((PHASE_CHUNKS, CHUNK), jnp.int32),
          pltpu.VMEM((PHASE_CHUNKS, CHUNK), jnp.int32),
          pltpu.VMEM((NBUF, CHUNK, D), jnp.float32),
          pltpu.VMEM_SHARED((ACC_ROWS, D), jnp.float32),
          pltpu.SemaphoreType.DMA((NBUF,)),
          pltpu.SemaphoreType.DMA((NBUF,)),
      ],
  )
  return kern(x, src_blocks, dst_blocks, zeros)


def _mm_body(p_ref, x_ref, w_ref, ws_ref, o_ref):
  agg = p_ref[0]
  for i in range(1, NC):
    agg = agg + p_ref[i]
  o_ref[...] = (jnp.dot(agg, w_ref[...], preferred_element_type=jnp.float32)
                + jnp.dot(x_ref[...], ws_ref[...], preferred_element_type=jnp.float32))


def _finish(partial, x, weight, weight_self):
  blk = 1000
  grid = (N_NODES // blk,)
  return pl.pallas_call(
      _mm_body,
      grid=grid,
      in_specs=[
          pl.BlockSpec((NC, blk, D), lambda i: (0, i, 0)),
          pl.BlockSpec((blk, D), lambda i: (i, 0)),
          pl.BlockSpec((D, D), lambda i: (0, 0)),
          pl.BlockSpec((D, D), lambda i: (0, 0)),
      ],
      out_specs=pl.BlockSpec((blk, D), lambda i: (i, 0)),
      out_shape=jax.ShapeDtypeStruct((N_NODES, D), jnp.float32),
  )(partial, x, weight, weight_self)


@jax.jit
def kernel(x, edge_index, weight, weight_self):
  n_edges = edge_index.shape[1]
  pad = E_PAD - n_edges
  pad_idx = jnp.arange(pad, dtype=jnp.int32)
  pads = jnp.stack([pad_idx % N_NODES, N_NODES + pad_idx % N_DUMMY])
  both = jnp.concatenate([edge_index, pads], axis=1)
  src_blocks = both[0].reshape(NC * NS, CORE_CHUNKS[0], CHUNK)
  dst_blocks = both[1].reshape(NC * NS, CORE_CHUNKS[0], CHUNK)
  zeros = jnp.zeros((ACC_ROWS, D), jnp.float32)

  partial = _segment_accumulate(x, src_blocks, dst_blocks, zeros)
  return _finish(partial, x, weight, weight_self)


# speedup vs baseline: 4.1293x; 1.0427x over previous
import jax
import jax.numpy as jnp
from jax import lax
from jax.experimental import pallas as pl
from jax.experimental.pallas import tpu as pltpu
from jax.experimental.pallas import tpu_sc as plsc

N_NODES = 10000
D = 128

NC = 2
NS = 16
CHUNK = 120
PHASE_CHUNKS = 8
N_PHASES = 11
CHUNKS_PER_TILE = PHASE_CHUNKS * N_PHASES
NBUF = 3
E_PAD = NC * NS * CHUNK * CHUNKS_PER_TILE
ROWS_PER_TILE = 632
ACC_ROWS = NS * ROWS_PER_TILE
LAST_ROWS = N_NODES - (NS - 1) * ROWS_PER_TILE
N_DUMMY = 112


def _sc_kernel_body(x_hbm, src_hbm, dst_hbm, zeros_hbm, out_hbm,
                    src_v, dst_v, ring, acc, gsem, ssem, isem):
  c = lax.axis_index("c")
  s = lax.axis_index("s")
  wid = c * NS + s

  slab = pl.ds(s * ROWS_PER_TILE, ROWS_PER_TILE)
  pltpu.sync_copy(zeros_hbm.at[slab], acc.at[slab])
  plsc.subcore_barrier()

  def gather_start(ib, ci, b):
    pltpu.async_copy(x_hbm.at[src_v.at[ib, ci]], ring.at[b], gsem.at[b])

  def gather_wait(ib, ci, b):
    pltpu.make_async_copy(
        x_hbm.at[src_v.at[ib, ci]], ring.at[b], gsem.at[b]).wait()

  def scatter_start(ib, ci, b):
    pltpu.async_copy(ring.at[b], acc.at[dst_v.at[ib, ci]], ssem.at[b], add=True)

  def scatter_wait(ib, ci, b):
    pltpu.make_async_copy(
        ring.at[b], acc.at[dst_v.at[ib, ci]], ssem.at[b]).wait()

  def idx_start(phase):
    ib = phase % 2
    pblk = pl.ds(phase * PHASE_CHUNKS, PHASE_CHUNKS)
    pltpu.async_copy(src_hbm.at[wid, pblk], src_v.at[ib], isem.at[ib])
    pltpu.async_copy(dst_hbm.at[wid, pblk], dst_v.at[ib], isem.at[ib])

  def idx_wait(phase):
    ib = phase % 2
    pblk = pl.ds(phase * PHASE_CHUNKS, PHASE_CHUNKS)
    pltpu.make_async_copy(src_hbm.at[wid, pblk], src_v.at[ib], isem.at[ib]).wait()
    pltpu.make_async_copy(dst_hbm.at[wid, pblk], dst_v.at[ib], isem.at[ib]).wait()

  idx_start(0)
  for phase in range(N_PHASES):
    ib = phase % 2
    idx_wait(phase)
    if phase + 1 < N_PHASES:
      idx_start(phase + 1)

    for b in range(NBUF):
      gather_start(ib, b, b)
    for ci in range(PHASE_CHUNKS):
      b = ci % NBUF
      gather_wait(ib, ci, b)
      scatter_start(ib, ci, b)
      p = ci - 1
      if p >= 0 and p + NBUF < PHASE_CHUNKS:
        scatter_wait(ib, p, p % NBUF)
        gather_start(ib, p + NBUF, p % NBUF)
    for ci in range(PHASE_CHUNKS - NBUF, PHASE_CHUNKS):
      if ci >= 0:
        scatter_wait(ib, ci, ci % NBUF)

  plsc.subcore_barrier()

  @pl.when(s < NS - 1)
  def _writeout_full():
    pltpu.sync_copy(acc.at[slab], out_hbm.at[c, slab])

  @pl.when(s == NS - 1)
  def _writeout_last():
    short = pl.ds((NS - 1) * ROWS_PER_TILE, LAST_ROWS)
    pltpu.sync_copy(acc.at[short], out_hbm.at[c, short])


def _segment_accumulate(x, src_blocks, dst_blocks, zeros):
  mesh = plsc.VectorSubcoreMesh(
      core_axis_name="c", subcore_axis_name="s", num_cores=NC, num_subcores=NS)
  kern = pl.kernel(
      _sc_kernel_body,
      out_type=jax.ShapeDtypeStruct((NC, N_NODES, D), jnp.float32),
      mesh=mesh,
      scratch_types=[
          pltpu.VMEM((2, PHASE_CHUNKS, CHUNK), jnp.int32),
          pltpu.VMEM((2, PHASE_CHUNKS, CHUNK), jnp.int32),
          pltpu.VMEM((NBUF, CHUNK, D), jnp.float32),
          pltpu.VMEM_SHARED((ACC_ROWS, D), jnp.float32),
          pltpu.SemaphoreType.DMA((NBUF,)),
          pltpu.SemaphoreType.DMA((NBUF,)),
          pltpu.SemaphoreType.DMA((2,)),
      ],
  )
  return kern(x, src_blocks, dst_blocks, zeros)


def _mm_body(p_ref, x_ref, w_ref, ws_ref, o_ref):
  agg = p_ref[0]
  for i in range(1, NC):
    agg = agg + p_ref[i]
  o_ref[...] = (jnp.dot(agg, w_ref[...], preferred_element_type=jnp.float32)
                + jnp.dot(x_ref[...], ws_ref[...], preferred_element_type=jnp.float32))


def _finish(partial, x, weight, weight_self):
  blk = 1000
  grid = (N_NODES // blk,)
  return pl.pallas_call(
      _mm_body,
      grid=grid,
      in_specs=[
          pl.BlockSpec((NC, blk, D), lambda i: (0, i, 0)),
          pl.BlockSpec((blk, D), lambda i: (i, 0)),
          pl.BlockSpec((D, D), lambda i: (0, 0)),
          pl.BlockSpec((D, D), lambda i: (0, 0)),
      ],
      out_specs=pl.BlockSpec((blk, D), lambda i: (i, 0)),
      out_shape=jax.ShapeDtypeStruct((N_NODES, D), jnp.float32),
  )(partial, x, weight, weight_self)


@jax.jit
def kernel(x, edge_index, weight, weight_self):
  n_edges = edge_index.shape[1]
  pad = E_PAD - n_edges
  pad_idx = jnp.arange(pad, dtype=jnp.int32)
  pads = jnp.stack([pad_idx % N_NODES, N_NODES + pad_idx % N_DUMMY])
  both = jnp.concatenate([edge_index, pads], axis=1)
  src_blocks = both[0].reshape(NC * NS, CHUNKS_PER_TILE, CHUNK)
  dst_blocks = both[1].reshape(NC * NS, CHUNKS_PER_TILE, CHUNK)
  zeros = jnp.zeros((ACC_ROWS, D), jnp.float32)

  partial = _segment_accumulate(x, src_blocks, dst_blocks, zeros)
  return _finish(partial, x, weight, weight_self)


# speedup vs baseline: 4.1855x; 1.0136x over previous
import jax
import jax.numpy as jnp
from jax import lax
from jax.experimental import pallas as pl
from jax.experimental.pallas import tpu as pltpu
from jax.experimental.pallas import tpu_sc as plsc

N_NODES = 10000
N_EDGES = 320000
D = 128

NC = 2
NS = 16
CHUNK = 125
PHASE_CHUNKS = 16
N_PHASES = 5
CHUNKS_PER_TILE = PHASE_CHUNKS * N_PHASES
NBUF = 2
ROWS_PER_TILE = 632
ACC_ROWS = NS * ROWS_PER_TILE
LAST_ROWS = N_NODES - (NS - 1) * ROWS_PER_TILE


def _sc_kernel_body(x_hbm, edges_hbm, zeros_hbm, out_hbm,
                    src_v, dst_v, ring, acc, gsem, ssem, isem):
  c = lax.axis_index("c")
  s = lax.axis_index("s")
  wid = c * NS + s

  slab = pl.ds(s * ROWS_PER_TILE, ROWS_PER_TILE)
  pltpu.sync_copy(zeros_hbm.at[slab], acc.at[slab])
  plsc.subcore_barrier()

  def gather_start(ib, ci, b):
    pltpu.async_copy(x_hbm.at[src_v.at[ib, ci]], ring.at[b], gsem.at[b])

  def gather_wait(ib, ci, b):
    pltpu.make_async_copy(
        x_hbm.at[src_v.at[ib, ci]], ring.at[b], gsem.at[b]).wait()

  def scatter_start(ib, ci, b):
    pltpu.async_copy(ring.at[b], acc.at[dst_v.at[ib, ci]], ssem.at[b], add=True)

  def scatter_wait(ib, ci, b):
    pltpu.make_async_copy(
        ring.at[b], acc.at[dst_v.at[ib, ci]], ssem.at[b]).wait()

  def idx_start(phase):
    ib = phase % 2
    pblk = pl.ds(phase * PHASE_CHUNKS, PHASE_CHUNKS)
    pltpu.async_copy(edges_hbm.at[0, wid, pblk], src_v.at[ib], isem.at[ib])
    pltpu.async_copy(edges_hbm.at[1, wid, pblk], dst_v.at[ib], isem.at[ib])

  def idx_wait(phase):
    ib = phase % 2
    pblk = pl.ds(phase * PHASE_CHUNKS, PHASE_CHUNKS)
    pltpu.make_async_copy(
        edges_hbm.at[0, wid, pblk], src_v.at[ib], isem.at[ib]).wait()
    pltpu.make_async_copy(
        edges_hbm.at[1, wid, pblk], dst_v.at[ib], isem.at[ib]).wait()

  idx_start(0)
  for phase in range(N_PHASES):
    ib = phase % 2
    idx_wait(phase)
    if phase + 1 < N_PHASES:
      idx_start(phase + 1)

    for b in range(NBUF):
      gather_start(ib, b, b)
    for g in range(PHASE_CHUNKS // NBUF):
      base = g * NBUF
      for b in range(NBUF):
        gather_wait(ib, base + b, b)
        scatter_start(ib, base + b, b)
      for b in range(NBUF):
        scatter_wait(ib, base + b, b)
        if base + NBUF + b < PHASE_CHUNKS:
          gather_start(ib, base + NBUF + b, b)

  plsc.subcore_barrier()

  @pl.when(s < NS - 1)
  def _writeout_full():
    pltpu.sync_copy(acc.at[slab], out_hbm.at[c, slab])

  @pl.when(s == NS - 1)
  def _writeout_last():
    short = pl.ds((NS - 1) * ROWS_PER_TILE, LAST_ROWS)
    pltpu.sync_copy(acc.at[short], out_hbm.at[c, short])


def _segment_accumulate(x, edge_blocks, zeros):
  mesh = plsc.VectorSubcoreMesh(
      core_axis_name="c", subcore_axis_name="s", num_cores=NC, num_subcores=NS)
  kern = pl.kernel(
      _sc_kernel_body,
      out_type=jax.ShapeDtypeStruct((NC, N_NODES, D), jnp.float32),
      mesh=mesh,
      scratch_types=[
          pltpu.VMEM((2, PHASE_CHUNKS, CHUNK), jnp.int32),
          pltpu.VMEM((2, PHASE_CHUNKS, CHUNK), jnp.int32),
          pltpu.VMEM((NBUF, CHUNK, D), jnp.float32),
          pltpu.VMEM_SHARED((ACC_ROWS, D), jnp.float32),
          pltpu.SemaphoreType.DMA((NBUF,)),
          pltpu.SemaphoreType.DMA((NBUF,)),
          pltpu.SemaphoreType.DMA((2,)),
      ],
  )
  return kern(x, edge_blocks, zeros)


def _mm_body(p_ref, x_ref, w_ref, ws_ref, o_ref):
  agg = p_ref[0]
  for i in range(1, NC):
    agg = agg + p_ref[i]
  o_ref[...] = (jnp.dot(agg, w_ref[...], preferred_element_type=jnp.float32)
                + jnp.dot(x_ref[...], ws_ref[...], preferred_element_type=jnp.float32))


def _finish(partial, x, weight, weight_self):
  blk = 1000
  grid = (N_NODES // blk,)
  return pl.pallas_call(
      _mm_body,
      grid=grid,
      in_specs=[
          pl.BlockSpec((NC, blk, D), lambda i: (0, i, 0)),
          pl.BlockSpec((blk, D), lambda i: (i, 0)),
          pl.BlockSpec((D, D), lambda i: (0, 0)),
          pl.BlockSpec((D, D), lambda i: (0, 0)),
      ],
      out_specs=pl.BlockSpec((blk, D), lambda i: (i, 0)),
      out_shape=jax.ShapeDtypeStruct((N_NODES, D), jnp.float32),
  )(partial, x, weight, weight_self)


@jax.jit
def kernel(x, edge_index, weight, weight_self):
  edge_blocks = edge_index.reshape(2, NC * NS, CHUNKS_PER_TILE, CHUNK)
  zeros = jnp.zeros((ACC_ROWS, D), jnp.float32)

  partial = _segment_accumulate(x, edge_blocks, zeros)
  return _finish(partial, x, weight, weight_self)


# speedup vs baseline: 4.1867x; 1.0003x over previous
import jax
import jax.numpy as jnp
from jax import lax
from jax.experimental import pallas as pl
from jax.experimental.pallas import tpu as pltpu
from jax.experimental.pallas import tpu_sc as plsc

N_NODES = 10000
N_EDGES = 320000
D = 128

NC = 2
NS = 16
CHUNK = 125
PHASE_CHUNKS = 16
N_PHASES = 5
CHUNKS_PER_TILE = PHASE_CHUNKS * N_PHASES
NBUF = 2
ROWS_PER_TILE = 632
ACC_ROWS = N_NODES
LAST_ROWS = N_NODES - (NS - 1) * ROWS_PER_TILE


def _sc_kernel_body(x_hbm, edges_hbm, zeros_hbm, out_hbm,
                    src_v, dst_v, ring, acc, gsem, ssem, isem):
  c = lax.axis_index("c")
  s = lax.axis_index("s")
  wid = c * NS + s

  slab = pl.ds(s * ROWS_PER_TILE, ROWS_PER_TILE)
  short = pl.ds((NS - 1) * ROWS_PER_TILE, LAST_ROWS)

  @pl.when(s < NS - 1)
  def _init_full():
    pltpu.sync_copy(zeros_hbm.at[slab], acc.at[slab])

  @pl.when(s == NS - 1)
  def _init_last():
    pltpu.sync_copy(zeros_hbm.at[short], acc.at[short])
  plsc.subcore_barrier()

  def gather_start(ib, ci, b):
    pltpu.async_copy(x_hbm.at[src_v.at[ib, ci]], ring.at[b], gsem.at[b])

  def gather_wait(ib, ci, b):
    pltpu.make_async_copy(
        x_hbm.at[src_v.at[ib, ci]], ring.at[b], gsem.at[b]).wait()

  def scatter_start(ib, ci, b):
    pltpu.async_copy(ring.at[b], acc.at[dst_v.at[ib, ci]], ssem.at[b], add=True)

  def scatter_wait(ib, ci, b):
    pltpu.make_async_copy(
        ring.at[b], acc.at[dst_v.at[ib, ci]], ssem.at[b]).wait()

  def idx_start(phase):
    ib = phase % 2
    pblk = pl.ds(phase * PHASE_CHUNKS, PHASE_CHUNKS)
    pltpu.async_copy(edges_hbm.at[0, wid, pblk], src_v.at[ib], isem.at[ib])
    pltpu.async_copy(edges_hbm.at[1, wid, pblk], dst_v.at[ib], isem.at[ib])

  def idx_wait(phase):
    ib = phase % 2
    pblk = pl.ds(phase * PHASE_CHUNKS, PHASE_CHUNKS)
    pltpu.make_async_copy(
        edges_hbm.at[0, wid, pblk], src_v.at[ib], isem.at[ib]).wait()
    pltpu.make_async_copy(
        edges_hbm.at[1, wid, pblk], dst_v.at[ib], isem.at[ib]).wait()

  idx_start(0)
  for phase in range(N_PHASES):
    ib = phase % 2
    idx_wait(phase)
    if phase + 1 < N_PHASES:
      idx_start(phase + 1)

    for b in range(NBUF):
      gather_start(ib, b, b)
    for g in range(PHASE_CHUNKS // NBUF):
      base = g * NBUF
      for b in range(NBUF):
        gather_wait(ib, base + b, b)
        scatter_start(ib, base + b, b)
      for b in range(NBUF):
        scatter_wait(ib, base + b, b)
        if base + NBUF + b < PHASE_CHUNKS:
          gather_start(ib, base + NBUF + b, b)

  plsc.subcore_barrier()

  @pl.when(s < NS - 1)
  def _writeout_full():
    pltpu.sync_copy(acc.at[slab], out_hbm.at[c, slab])

  @pl.when(s == NS - 1)
  def _writeout_last():
    pltpu.sync_copy(acc.at[short], out_hbm.at[c, short])


def _segment_accumulate(x, edge_blocks, zeros):
  mesh = plsc.VectorSubcoreMesh(
      core_axis_name="c", subcore_axis_name="s", num_cores=NC, num_subcores=NS)
  kern = pl.kernel(
      _sc_kernel_body,
      out_type=jax.ShapeDtypeStruct((NC, N_NODES, D), jnp.float32),
      mesh=mesh,
      scratch_types=[
          pltpu.VMEM((2, PHASE_CHUNKS, CHUNK), jnp.int32),
          pltpu.VMEM((2, PHASE_CHUNKS, CHUNK), jnp.int32),
          pltpu.VMEM((NBUF, CHUNK, D), jnp.float32),
          pltpu.VMEM_SHARED((ACC_ROWS, D), jnp.float32),
          pltpu.SemaphoreType.DMA((NBUF,)),
          pltpu.SemaphoreType.DMA((NBUF,)),
          pltpu.SemaphoreType.DMA((2,)),
      ],
  )
  return kern(x, edge_blocks, zeros)


def _mm_body(p_ref, x_ref, w_ref, ws_ref, o_ref):
  agg = p_ref[0]
  for i in range(1, NC):
    agg = agg + p_ref[i]
  o_ref[...] = (jnp.dot(agg, w_ref[...], preferred_element_type=jnp.float32)
                + jnp.dot(x_ref[...], ws_ref[...], preferred_element_type=jnp.float32))


def _finish(partial, x, weight, weight_self):
  blk = 1000
  grid = (N_NODES // blk,)
  return pl.pallas_call(
      _mm_body,
      grid=grid,
      in_specs=[
          pl.BlockSpec((NC, blk, D), lambda i: (0, i, 0)),
          pl.BlockSpec((blk, D), lambda i: (i, 0)),
          pl.BlockSpec((D, D), lambda i: (0, 0)),
          pl.BlockSpec((D, D), lambda i: (0, 0)),
      ],
      out_specs=pl.BlockSpec((blk, D), lambda i: (i, 0)),
      out_shape=jax.ShapeDtypeStruct((N_NODES, D), jnp.float32),
  )(partial, x, weight, weight_self)


@jax.jit
def kernel(x, edge_index, weight, weight_self):
  edge_blocks = edge_index.reshape(2, NC * NS, CHUNKS_PER_TILE, CHUNK)
  zeros = jnp.zeros((ACC_ROWS, D), jnp.float32)

  partial = _segment_accumulate(x, edge_blocks, zeros)
  return _finish(partial, x, weight, weight_self)
